# baseline (device time: 351303 ns/iter reference)
import jax
import jax.numpy as jnp
from jax import lax
from jax.experimental import pallas as pl
from jax.experimental.pallas import tpu as pltpu

M = 4096
D = 4096
R = M // 4
CH = 128
NC = R // CH
NF = NC // 2


def _body(x_ref, resid_ref, g_ref, out_ref,
          rx_vmem, xc_vmem, rc_vmem, oc_vmem,
          sx_send, sx_recv, sy_send, sy_recv, sz_send, sz_recv,
          sfy_send, sfy_recv, sfz_send, sfz_recv, lsem):
    my_x = lax.axis_index("x")
    my_y = lax.axis_index("y")
    my_z = lax.axis_index("z")
    x_peer = (1 - my_x, my_y, my_z)
    y_peer = (my_x, 1 - my_y, my_z)
    z_peer = (my_x, my_y, 1 - my_z)

    q_me = 2 * my_y + my_z
    q_a = 2 * (1 - my_y) + my_z
    q_b = 2 * my_y + (1 - my_z)

    def rows(q, c):
        return pl.ds(q * R + c * CH, CH)

    barrier_sem = pltpu.get_barrier_semaphore()
    for peer in (x_peer, y_peer, z_peer):
        pl.semaphore_signal(barrier_sem, inc=1, device_id=peer,
                            device_id_type=pl.DeviceIdType.MESH)
    pl.semaphore_wait(barrier_sem, 3)

    rdma_x = []
    for c in range(NC):
        r = pltpu.make_async_remote_copy(
            src_ref=x_ref.at[rows(q_me, c)],
            dst_ref=rx_vmem.at[c],
            send_sem=sx_send.at[c],
            recv_sem=sx_recv.at[c],
            device_id=x_peer,
            device_id_type=pl.DeviceIdType.MESH,
        )
        r.start()
        rdma_x.append(r)

    rdma_y = []
    rdma_z = []
    fwd = [None] * NC

    def issue_fwd(h):
        if h % 2 == 0:
            rdma_y[h].wait_recv()
            r = pltpu.make_async_remote_copy(
                src_ref=out_ref.at[rows(q_a, h)],
                dst_ref=out_ref.at[rows(q_a, h)],
                send_sem=sfz_send.at[h // 2],
                recv_sem=sfz_recv.at[h // 2],
                device_id=z_peer,
                device_id_type=pl.DeviceIdType.MESH,
            )
        else:
            rdma_z[h].wait_recv()
            r = pltpu.make_async_remote_copy(
                src_ref=out_ref.at[rows(q_b, h)],
                dst_ref=out_ref.at[rows(q_b, h)],
                send_sem=sfy_send.at[h // 2],
                recv_sem=sfy_recv.at[h // 2],
                device_id=y_peer,
                device_id_type=pl.DeviceIdType.MESH,
            )
        r.start()
        fwd[h] = r

    for c in range(NC):
        cp_x = pltpu.make_async_copy(x_ref.at[rows(q_me, c)], xc_vmem,
                                     lsem.at[0])
        cp_r = pltpu.make_async_copy(resid_ref.at[rows(q_me, c)], rc_vmem,
                                     lsem.at[1])
        cp_x.start()
        cp_r.start()
        rdma_x[c].wait_recv()
        cp_x.wait()
        cp_r.wait()

        y = xc_vmem[...] + rx_vmem[c] + rc_vmem[...]
        ms = jnp.mean(y * y, axis=-1, keepdims=True)
        oc_vmem[...] = y * lax.rsqrt(ms + 1e-6) * g_ref[...]

        cp_o = pltpu.make_async_copy(oc_vmem, out_ref.at[rows(q_me, c)],
                                     lsem.at[2])
        cp_o.start()
        cp_o.wait()

        for peer, sends, recvs, acc in (
            (y_peer, sy_send, sy_recv, rdma_y),
            (z_peer, sz_send, sz_recv, rdma_z),
        ):
            r = pltpu.make_async_remote_copy(
                src_ref=out_ref.at[rows(q_me, c)],
                dst_ref=out_ref.at[rows(q_me, c)],
                send_sem=sends.at[c],
                recv_sem=recvs.at[c],
                device_id=peer,
                device_id_type=pl.DeviceIdType.MESH,
            )
            r.start()
            acc.append(r)

        if c >= 2:
            issue_fwd(c - 2)

    for h in (NC - 2, NC - 1):
        issue_fwd(h)

    for c in range(1, NC, 2):
        rdma_y[c].wait_recv()
    for c in range(0, NC, 2):
        rdma_z[c].wait_recv()
    for r in fwd:
        r.wait_recv()

    for r in rdma_x + rdma_y + rdma_z + fwd:
        r.wait_send()


def kernel(partial, resid, gamma):
    x2d = partial.reshape(M, D)
    return pl.pallas_call(
        _body,
        out_shape=jax.ShapeDtypeStruct((M, D), jnp.float32),
        in_specs=[
            pl.BlockSpec(memory_space=pl.ANY),
            pl.BlockSpec(memory_space=pl.ANY),
            pl.BlockSpec(memory_space=pltpu.VMEM),
        ],
        out_specs=pl.BlockSpec(memory_space=pl.ANY),
        scratch_shapes=[
            pltpu.VMEM((NC, CH, D), jnp.float32),
            pltpu.VMEM((CH, D), jnp.float32),
            pltpu.VMEM((CH, D), jnp.float32),
            pltpu.VMEM((CH, D), jnp.float32),
            pltpu.SemaphoreType.DMA((NC,)),
            pltpu.SemaphoreType.DMA((NC,)),
            pltpu.SemaphoreType.DMA((NC,)),
            pltpu.SemaphoreType.DMA((NC,)),
            pltpu.SemaphoreType.DMA((NC,)),
            pltpu.SemaphoreType.DMA((NC,)),
            pltpu.SemaphoreType.DMA((NF,)),
            pltpu.SemaphoreType.DMA((NF,)),
            pltpu.SemaphoreType.DMA((NF,)),
            pltpu.SemaphoreType.DMA((NF,)),
            pltpu.SemaphoreType.DMA((3,)),
        ],
        compiler_params=pltpu.CompilerParams(
            collective_id=0,
            vmem_limit_bytes=64 * 1024 * 1024,
        ),
    )(x2d, resid, gamma.reshape(1, D))


# device time: 339523 ns/iter; 1.0347x vs baseline; 1.0347x over previous
import jax
import jax.numpy as jnp
from jax import lax
from jax.experimental import pallas as pl
from jax.experimental.pallas import tpu as pltpu

M = 4096
D = 4096
R = M // 4
CH = 64
NC = R // CH
NF = NC // 2


def _body(x_ref, resid_ref, g_ref, out_ref,
          rx_vmem, xc_vmem, rc_vmem, oc_vmem,
          sx_send, sx_recv, sy_send, sy_recv, sz_send, sz_recv,
          sfy_send, sfy_recv, sfz_send, sfz_recv, lsem):
    my_x = lax.axis_index("x")
    my_y = lax.axis_index("y")
    my_z = lax.axis_index("z")
    x_peer = (1 - my_x, my_y, my_z)
    y_peer = (my_x, 1 - my_y, my_z)
    z_peer = (my_x, my_y, 1 - my_z)

    q_me = 2 * my_y + my_z
    q_a = 2 * (1 - my_y) + my_z
    q_b = 2 * my_y + (1 - my_z)

    def rows(q, c):
        return pl.ds(q * R + c * CH, CH)

    barrier_sem = pltpu.get_barrier_semaphore()
    for peer in (x_peer, y_peer, z_peer):
        pl.semaphore_signal(barrier_sem, inc=1, device_id=peer,
                            device_id_type=pl.DeviceIdType.MESH)
    pl.semaphore_wait(barrier_sem, 3)

    rdma_x = []
    for c in range(NC):
        r = pltpu.make_async_remote_copy(
            src_ref=x_ref.at[rows(q_me, c)],
            dst_ref=rx_vmem.at[c],
            send_sem=sx_send.at[c],
            recv_sem=sx_recv.at[c],
            device_id=x_peer,
            device_id_type=pl.DeviceIdType.MESH,
        )
        r.start()
        rdma_x.append(r)

    rdma_y = []
    rdma_z = []
    fwd = [None] * NC

    def issue_fwd(h):
        if h % 2 == 0:
            rdma_y[h].wait_recv()
            r = pltpu.make_async_remote_copy(
                src_ref=out_ref.at[rows(q_a, h)],
                dst_ref=out_ref.at[rows(q_a, h)],
                send_sem=sfz_send.at[h // 2],
                recv_sem=sfz_recv.at[h // 2],
                device_id=z_peer,
                device_id_type=pl.DeviceIdType.MESH,
            )
        else:
            rdma_z[h].wait_recv()
            r = pltpu.make_async_remote_copy(
                src_ref=out_ref.at[rows(q_b, h)],
                dst_ref=out_ref.at[rows(q_b, h)],
                send_sem=sfy_send.at[h // 2],
                recv_sem=sfy_recv.at[h // 2],
                device_id=y_peer,
                device_id_type=pl.DeviceIdType.MESH,
            )
        r.start()
        fwd[h] = r

    for c in range(NC):
        cp_x = pltpu.make_async_copy(x_ref.at[rows(q_me, c)], xc_vmem,
                                     lsem.at[0])
        cp_r = pltpu.make_async_copy(resid_ref.at[rows(q_me, c)], rc_vmem,
                                     lsem.at[1])
        cp_x.start()
        cp_r.start()
        rdma_x[c].wait_recv()
        cp_x.wait()
        cp_r.wait()

        y = xc_vmem[...] + rx_vmem[c] + rc_vmem[...]
        ms = jnp.mean(y * y, axis=-1, keepdims=True)
        oc_vmem[...] = y * lax.rsqrt(ms + 1e-6) * g_ref[...]

        cp_o = pltpu.make_async_copy(oc_vmem, out_ref.at[rows(q_me, c)],
                                     lsem.at[2])
        cp_o.start()
        cp_o.wait()

        for peer, sends, recvs, acc in (
            (y_peer, sy_send, sy_recv, rdma_y),
            (z_peer, sz_send, sz_recv, rdma_z),
        ):
            r = pltpu.make_async_remote_copy(
                src_ref=out_ref.at[rows(q_me, c)],
                dst_ref=out_ref.at[rows(q_me, c)],
                send_sem=sends.at[c],
                recv_sem=recvs.at[c],
                device_id=peer,
                device_id_type=pl.DeviceIdType.MESH,
            )
            r.start()
            acc.append(r)

        if c >= 2:
            issue_fwd(c - 2)

    for h in (NC - 2, NC - 1):
        issue_fwd(h)

    for c in range(1, NC, 2):
        rdma_y[c].wait_recv()
    for c in range(0, NC, 2):
        rdma_z[c].wait_recv()
    for r in fwd:
        r.wait_recv()

    for r in rdma_x + rdma_y + rdma_z + fwd:
        r.wait_send()


def kernel(partial, resid, gamma):
    x2d = partial.reshape(M, D)
    return pl.pallas_call(
        _body,
        out_shape=jax.ShapeDtypeStruct((M, D), jnp.float32),
        in_specs=[
            pl.BlockSpec(memory_space=pl.ANY),
            pl.BlockSpec(memory_space=pl.ANY),
            pl.BlockSpec(memory_space=pltpu.VMEM),
        ],
        out_specs=pl.BlockSpec(memory_space=pl.ANY),
        scratch_shapes=[
            pltpu.VMEM((NC, CH, D), jnp.float32),
            pltpu.VMEM((CH, D), jnp.float32),
            pltpu.VMEM((CH, D), jnp.float32),
            pltpu.VMEM((CH, D), jnp.float32),
            pltpu.SemaphoreType.DMA((NC,)),
            pltpu.SemaphoreType.DMA((NC,)),
            pltpu.SemaphoreType.DMA((NC,)),
            pltpu.SemaphoreType.DMA((NC,)),
            pltpu.SemaphoreType.DMA((NC,)),
            pltpu.SemaphoreType.DMA((NC,)),
            pltpu.SemaphoreType.DMA((NF,)),
            pltpu.SemaphoreType.DMA((NF,)),
            pltpu.SemaphoreType.DMA((NF,)),
            pltpu.SemaphoreType.DMA((NF,)),
            pltpu.SemaphoreType.DMA((3,)),
        ],
        compiler_params=pltpu.CompilerParams(
            collective_id=0,
            vmem_limit_bytes=64 * 1024 * 1024,
        ),
    )(x2d, resid, gamma.reshape(1, D))


# device time: 333349 ns/iter; 1.0539x vs baseline; 1.0185x over previous
import jax
import jax.numpy as jnp
from jax import lax
from jax.experimental import pallas as pl
from jax.experimental.pallas import tpu as pltpu

M = 4096
D = 4096
R = M // 4
CH = 32
NC = R // CH
NF = NC // 2


def _body(x_ref, resid_ref, g_ref, out_ref,
          rx_vmem, xc_vmem, rc_vmem, oc_vmem,
          sx_send, sx_recv, sy_send, sy_recv, sz_send, sz_recv,
          sfy_send, sfy_recv, sfz_send, sfz_recv, lsem):
    my_x = lax.axis_index("x")
    my_y = lax.axis_index("y")
    my_z = lax.axis_index("z")
    x_peer = (1 - my_x, my_y, my_z)
    y_peer = (my_x, 1 - my_y, my_z)
    z_peer = (my_x, my_y, 1 - my_z)

    q_me = 2 * my_y + my_z
    q_a = 2 * (1 - my_y) + my_z
    q_b = 2 * my_y + (1 - my_z)

    def rows(q, c):
        return pl.ds(q * R + c * CH, CH)

    barrier_sem = pltpu.get_barrier_semaphore()
    for peer in (x_peer, y_peer, z_peer):
        pl.semaphore_signal(barrier_sem, inc=1, device_id=peer,
                            device_id_type=pl.DeviceIdType.MESH)
    pl.semaphore_wait(barrier_sem, 3)

    rdma_x = []
    for c in range(NC):
        r = pltpu.make_async_remote_copy(
            src_ref=x_ref.at[rows(q_me, c)],
            dst_ref=rx_vmem.at[c],
            send_sem=sx_send.at[c],
            recv_sem=sx_recv.at[c],
            device_id=x_peer,
            device_id_type=pl.DeviceIdType.MESH,
        )
        r.start()
        rdma_x.append(r)

    rdma_y = []
    rdma_z = []
    fwd = [None] * NC

    def issue_fwd(h):
        if h % 2 == 0:
            rdma_y[h].wait_recv()
            r = pltpu.make_async_remote_copy(
                src_ref=out_ref.at[rows(q_a, h)],
                dst_ref=out_ref.at[rows(q_a, h)],
                send_sem=sfz_send.at[h // 2],
                recv_sem=sfz_recv.at[h // 2],
                device_id=z_peer,
                device_id_type=pl.DeviceIdType.MESH,
            )
        else:
            rdma_z[h].wait_recv()
            r = pltpu.make_async_remote_copy(
                src_ref=out_ref.at[rows(q_b, h)],
                dst_ref=out_ref.at[rows(q_b, h)],
                send_sem=sfy_send.at[h // 2],
                recv_sem=sfy_recv.at[h // 2],
                device_id=y_peer,
                device_id_type=pl.DeviceIdType.MESH,
            )
        r.start()
        fwd[h] = r

    for c in range(NC):
        cp_x = pltpu.make_async_copy(x_ref.at[rows(q_me, c)], xc_vmem,
                                     lsem.at[0])
        cp_r = pltpu.make_async_copy(resid_ref.at[rows(q_me, c)], rc_vmem,
                                     lsem.at[1])
        cp_x.start()
        cp_r.start()
        rdma_x[c].wait_recv()
        cp_x.wait()
        cp_r.wait()

        y = xc_vmem[...] + rx_vmem[c] + rc_vmem[...]
        ms = jnp.mean(y * y, axis=-1, keepdims=True)
        oc_vmem[...] = y * lax.rsqrt(ms + 1e-6) * g_ref[...]

        cp_o = pltpu.make_async_copy(oc_vmem, out_ref.at[rows(q_me, c)],
                                     lsem.at[2])
        cp_o.start()
        cp_o.wait()

        for peer, sends, recvs, acc in (
            (y_peer, sy_send, sy_recv, rdma_y),
            (z_peer, sz_send, sz_recv, rdma_z),
        ):
            r = pltpu.make_async_remote_copy(
                src_ref=out_ref.at[rows(q_me, c)],
                dst_ref=out_ref.at[rows(q_me, c)],
                send_sem=sends.at[c],
                recv_sem=recvs.at[c],
                device_id=peer,
                device_id_type=pl.DeviceIdType.MESH,
            )
            r.start()
            acc.append(r)

        if c >= 2:
            issue_fwd(c - 2)

    for h in (NC - 2, NC - 1):
        issue_fwd(h)

    for c in range(1, NC, 2):
        rdma_y[c].wait_recv()
    for c in range(0, NC, 2):
        rdma_z[c].wait_recv()
    for r in fwd:
        r.wait_recv()

    for r in rdma_x + rdma_y + rdma_z + fwd:
        r.wait_send()


def kernel(partial, resid, gamma):
    x2d = partial.reshape(M, D)
    return pl.pallas_call(
        _body,
        out_shape=jax.ShapeDtypeStruct((M, D), jnp.float32),
        in_specs=[
            pl.BlockSpec(memory_space=pl.ANY),
            pl.BlockSpec(memory_space=pl.ANY),
            pl.BlockSpec(memory_space=pltpu.VMEM),
        ],
        out_specs=pl.BlockSpec(memory_space=pl.ANY),
        scratch_shapes=[
            pltpu.VMEM((NC, CH, D), jnp.float32),
            pltpu.VMEM((CH, D), jnp.float32),
            pltpu.VMEM((CH, D), jnp.float32),
            pltpu.VMEM((CH, D), jnp.float32),
            pltpu.SemaphoreType.DMA((NC,)),
            pltpu.SemaphoreType.DMA((NC,)),
            pltpu.SemaphoreType.DMA((NC,)),
            pltpu.SemaphoreType.DMA((NC,)),
            pltpu.SemaphoreType.DMA((NC,)),
            pltpu.SemaphoreType.DMA((NC,)),
            pltpu.SemaphoreType.DMA((NF,)),
            pltpu.SemaphoreType.DMA((NF,)),
            pltpu.SemaphoreType.DMA((NF,)),
            pltpu.SemaphoreType.DMA((NF,)),
            pltpu.SemaphoreType.DMA((3,)),
        ],
        compiler_params=pltpu.CompilerParams(
            collective_id=0,
            vmem_limit_bytes=64 * 1024 * 1024,
        ),
    )(x2d, resid, gamma.reshape(1, D))
